# baseline (speedup 1.0000x reference)
import functools

import jax
import jax.numpy as jnp
from jax import lax
from jax.experimental import pallas as pl
from jax.experimental.pallas import tpu as pltpu
from jax.experimental.pallas import tpu_sc as plsc


_NC = 2
_NS = 16
_NW = _NC * _NS
_ROWS_W = 3128
_CHUNK = 184
_NSLOT = 5
_SLAG = 3
_CROWS_W = 32


def _sc_relay(item_ref, cat_ref, oitem_ref, ocat_ref,
              bufs, cat_buf, in_sems, out_sems, cat_sem):
    n = item_ref.shape[0]
    nc = cat_ref.shape[0]
    wid = lax.axis_index("s") * _NC + lax.axis_index("c")
    start = jnp.minimum(wid * _ROWS_W, n - _ROWS_W)
    nchunk = _ROWS_W // _CHUNK

    cstart = jnp.minimum(wid * _CROWS_W, nc - _CROWS_W)
    pltpu.make_async_copy(
        cat_ref.at[pl.ds(cstart, _CROWS_W)], cat_buf, cat_sem).start()

    for idx in range(nchunk + _SLAG):
        slot = idx % _NSLOT
        if idx < nchunk:
            off = start + idx * _CHUNK
            if idx >= _NSLOT:
                poff = start + (idx - _NSLOT) * _CHUNK
                pltpu.make_async_copy(
                    bufs.at[slot], oitem_ref.at[pl.ds(poff, _CHUNK)],
                    out_sems.at[slot]).wait()
            pltpu.make_async_copy(
                item_ref.at[pl.ds(off, _CHUNK)], bufs.at[slot],
                in_sems.at[slot]).start()
        j = idx - _SLAG
        if 0 <= j < nchunk:
            jslot = j % _NSLOT
            joff = start + j * _CHUNK
            pltpu.make_async_copy(
                item_ref.at[pl.ds(joff, _CHUNK)], bufs.at[jslot],
                in_sems.at[jslot]).wait()
            pltpu.make_async_copy(
                bufs.at[jslot], oitem_ref.at[pl.ds(joff, _CHUNK)],
                out_sems.at[jslot]).start()

    pltpu.make_async_copy(
        cat_ref.at[pl.ds(cstart, _CROWS_W)], cat_buf, cat_sem).wait()
    pltpu.make_async_copy(
        cat_buf, ocat_ref.at[pl.ds(cstart, _CROWS_W)], cat_sem).start()

    for j in range(max(nchunk - _NSLOT, 0), nchunk):
        jslot = j % _NSLOT
        joff = start + j * _CHUNK
        pltpu.make_async_copy(
            bufs.at[jslot], oitem_ref.at[pl.ds(joff, _CHUNK)],
            out_sems.at[jslot]).wait()
    pltpu.make_async_copy(
        cat_buf, ocat_ref.at[pl.ds(cstart, _CROWS_W)], cat_sem).wait()


def _sc_copy(emb_item, emb_category):
    d = emb_item.shape[1]
    mesh = plsc.VectorSubcoreMesh(core_axis_name="c", subcore_axis_name="s")
    run = functools.partial(
        pl.kernel,
        out_type=(
            jax.ShapeDtypeStruct(emb_item.shape, emb_item.dtype),
            jax.ShapeDtypeStruct(emb_category.shape, emb_category.dtype),
        ),
        mesh=mesh,
        scratch_types=[
            pltpu.VMEM((_NSLOT, _CHUNK, d), jnp.float32),
            pltpu.VMEM((_CROWS_W, d), jnp.float32),
            pltpu.SemaphoreType.DMA((_NSLOT,)),
            pltpu.SemaphoreType.DMA((_NSLOT,)),
            pltpu.SemaphoreType.DMA,
        ],
    )(_sc_relay)
    return run(emb_item, emb_category)


_B = 5000
_NBUF = 8
_LAG = 4


def _tc_relay(u_ref, ou_ref, bufs, in_sems, out_sems):
    n = u_ref.shape[0]
    total = n // _B
    for idx in range(total + _LAG):
        slot = idx % _NBUF
        if idx < total:
            off = idx * _B
            if idx >= _NBUF:
                poff = (idx - _NBUF) * _B
                pltpu.make_async_copy(
                    bufs.at[slot], ou_ref.at[pl.ds(poff, _B)],
                    out_sems.at[slot]).wait()
            pltpu.make_async_copy(
                u_ref.at[pl.ds(off, _B)], bufs.at[slot],
                in_sems.at[slot]).start()
        j = idx - _LAG
        if 0 <= j < total:
            jslot = j % _NBUF
            joff = j * _B
            pltpu.make_async_copy(
                u_ref.at[pl.ds(joff, _B)], bufs.at[jslot],
                in_sems.at[jslot]).wait()
            pltpu.make_async_copy(
                bufs.at[jslot], ou_ref.at[pl.ds(joff, _B)],
                out_sems.at[jslot]).start()
    for j in range(max(total - _NBUF, 0), total):
        jslot = j % _NBUF
        joff = j * _B
        pltpu.make_async_copy(
            bufs.at[jslot], ou_ref.at[pl.ds(joff, _B)],
            out_sems.at[jslot]).wait()


def _tc_copy(emb_user):
    n, d = emb_user.shape
    any_spec = pl.BlockSpec(memory_space=pl.ANY)
    return pl.pallas_call(
        _tc_relay,
        out_shape=jax.ShapeDtypeStruct(emb_user.shape, emb_user.dtype),
        in_specs=[any_spec],
        out_specs=any_spec,
        scratch_shapes=[
            pltpu.VMEM((_NBUF, _B, d), jnp.float32),
            pltpu.SemaphoreType.DMA((_NBUF,)),
            pltpu.SemaphoreType.DMA((_NBUF,)),
        ],
    )(emb_user)


def kernel(emb_user, emb_item, emb_category):
    out_item, out_cat = _sc_copy(emb_item, emb_category)
    out_user = _tc_copy(emb_user)
    return (out_user, out_item, out_cat)


# speedup vs baseline: 1.0108x; 1.0108x over previous
import functools

import jax
import jax.numpy as jnp
from jax import lax
from jax.experimental import pallas as pl
from jax.experimental.pallas import tpu as pltpu
from jax.experimental.pallas import tpu_sc as plsc


_NC = 2
_NS = 16
_NW = _NC * _NS
_ROWS_W = 3128
_CHUNK = 184
_NSLOT = 4
_SLAG = 2
_CROWS_W = 32


def _sc_relay(item_ref, cat_ref, oitem_ref, ocat_ref,
              bufs, cat_buf, in_sems, out_sems, cat_sem):
    n = item_ref.shape[0]
    nc = cat_ref.shape[0]
    wid = lax.axis_index("s") * _NC + lax.axis_index("c")
    start = jnp.minimum(wid * _ROWS_W, n - _ROWS_W)
    nchunk = _ROWS_W // _CHUNK

    cstart = jnp.minimum(wid * _CROWS_W, nc - _CROWS_W)
    pltpu.make_async_copy(
        cat_ref.at[pl.ds(cstart, _CROWS_W)], cat_buf, cat_sem).start()

    for idx in range(nchunk + _SLAG):
        slot = idx % _NSLOT
        if idx < nchunk:
            off = start + idx * _CHUNK
            if idx >= _NSLOT:
                poff = start + (idx - _NSLOT) * _CHUNK
                pltpu.make_async_copy(
                    bufs.at[slot], oitem_ref.at[pl.ds(poff, _CHUNK)],
                    out_sems.at[slot]).wait()
            pltpu.make_async_copy(
                item_ref.at[pl.ds(off, _CHUNK)], bufs.at[slot],
                in_sems.at[slot]).start()
        j = idx - _SLAG
        if 0 <= j < nchunk:
            jslot = j % _NSLOT
            joff = start + j * _CHUNK
            pltpu.make_async_copy(
                item_ref.at[pl.ds(joff, _CHUNK)], bufs.at[jslot],
                in_sems.at[jslot]).wait()
            pltpu.make_async_copy(
                bufs.at[jslot], oitem_ref.at[pl.ds(joff, _CHUNK)],
                out_sems.at[jslot]).start()

    pltpu.make_async_copy(
        cat_ref.at[pl.ds(cstart, _CROWS_W)], cat_buf, cat_sem).wait()
    pltpu.make_async_copy(
        cat_buf, ocat_ref.at[pl.ds(cstart, _CROWS_W)], cat_sem).start()

    for j in range(max(nchunk - _NSLOT, 0), nchunk):
        jslot = j % _NSLOT
        joff = start + j * _CHUNK
        pltpu.make_async_copy(
            bufs.at[jslot], oitem_ref.at[pl.ds(joff, _CHUNK)],
            out_sems.at[jslot]).wait()
    pltpu.make_async_copy(
        cat_buf, ocat_ref.at[pl.ds(cstart, _CROWS_W)], cat_sem).wait()


def _sc_copy(emb_item, emb_category):
    d = emb_item.shape[1]
    mesh = plsc.VectorSubcoreMesh(core_axis_name="c", subcore_axis_name="s")
    run = functools.partial(
        pl.kernel,
        out_type=(
            jax.ShapeDtypeStruct(emb_item.shape, emb_item.dtype),
            jax.ShapeDtypeStruct(emb_category.shape, emb_category.dtype),
        ),
        mesh=mesh,
        scratch_types=[
            pltpu.VMEM((_NSLOT, _CHUNK, d), jnp.float32),
            pltpu.VMEM((_CROWS_W, d), jnp.float32),
            pltpu.SemaphoreType.DMA((_NSLOT,)),
            pltpu.SemaphoreType.DMA((_NSLOT,)),
            pltpu.SemaphoreType.DMA,
        ],
    )(_sc_relay)
    return run(emb_item, emb_category)


_B = 5000
_NBUF = 8
_LAG = 4


def _tc_relay(u_ref, ou_ref, bufs, in_sems, out_sems):
    n = u_ref.shape[0]
    total = n // _B
    for idx in range(total + _LAG):
        slot = idx % _NBUF
        if idx < total:
            off = idx * _B
            if idx >= _NBUF:
                poff = (idx - _NBUF) * _B
                pltpu.make_async_copy(
                    bufs.at[slot], ou_ref.at[pl.ds(poff, _B)],
                    out_sems.at[slot]).wait()
            pltpu.make_async_copy(
                u_ref.at[pl.ds(off, _B)], bufs.at[slot],
                in_sems.at[slot]).start()
        j = idx - _LAG
        if 0 <= j < total:
            jslot = j % _NBUF
            joff = j * _B
            pltpu.make_async_copy(
                u_ref.at[pl.ds(joff, _B)], bufs.at[jslot],
                in_sems.at[jslot]).wait()
            pltpu.make_async_copy(
                bufs.at[jslot], ou_ref.at[pl.ds(joff, _B)],
                out_sems.at[jslot]).start()
    for j in range(max(total - _NBUF, 0), total):
        jslot = j % _NBUF
        joff = j * _B
        pltpu.make_async_copy(
            bufs.at[jslot], ou_ref.at[pl.ds(joff, _B)],
            out_sems.at[jslot]).wait()


def _tc_copy(emb_user):
    n, d = emb_user.shape
    any_spec = pl.BlockSpec(memory_space=pl.ANY)
    return pl.pallas_call(
        _tc_relay,
        out_shape=jax.ShapeDtypeStruct(emb_user.shape, emb_user.dtype),
        in_specs=[any_spec],
        out_specs=any_spec,
        scratch_shapes=[
            pltpu.VMEM((_NBUF, _B, d), jnp.float32),
            pltpu.SemaphoreType.DMA((_NBUF,)),
            pltpu.SemaphoreType.DMA((_NBUF,)),
        ],
    )(emb_user)


def kernel(emb_user, emb_item, emb_category):
    out_item, out_cat = _sc_copy(emb_item, emb_category)
    out_user = _tc_copy(emb_user)
    return (out_user, out_item, out_cat)
